# Initial kernel scaffold; baseline (speedup 1.0000x reference)
import jax
import jax.numpy as jnp
from jax.experimental import pallas as pl


def kernel(x, edge_index, W1, b1, W2, b2, Wf, bf):
    raise NotImplementedError("write your pallas kernel here")


# speedup vs baseline: 2.0976x; 2.0976x over previous
import functools

import jax
import jax.numpy as jnp
from jax import lax
from jax.experimental import pallas as pl
from jax.experimental.pallas import tpu as pltpu
from jax.experimental.pallas import tpu_sc as plsc

NC = 2
NS = 16
LANE = 128
LC = 16


def _part(nitems, nworkers, wid):
    q, r = nitems // nworkers, nitems % nworkers
    base = wid * q + jnp.minimum(wid, r)
    cnt = q + (wid < r).astype(jnp.int32)
    return base, cnt


def _sc_mesh():
    return plsc.VectorSubcoreMesh(core_axis_name="c", subcore_axis_name="s")


def _deg_body(n2, nwin, ei3, out, acc, zb, dstbuf, onesv):
    c = lax.axis_index("c")
    s = lax.axis_index("s")
    wid = s * NC + c
    rt = n2 // NS
    rs = pl.multiple_of(s * rt, 512)

    def zfill(i, carry):
        zb[pl.ds(pl.multiple_of(i * 16, 16), 16)] = jnp.zeros((16,), jnp.float32)
        return carry

    lax.fori_loop(0, rt // 16, zfill, 0)
    for i in range(LANE // 16):
        onesv[pl.ds(i * 16, 16)] = jnp.ones((16,), jnp.float32)
    pltpu.sync_copy(zb, acc.at[pl.ds(rs, rt)])
    plsc.subcore_barrier()
    base, cnt = _part(nwin // LC, NC * NS, wid)

    def grp(g, carry):
        w0 = pl.multiple_of((base + g) * LC, LC)
        pltpu.sync_copy(ei3.at[1, pl.ds(w0, LC), :], dstbuf)
        for j in range(LC):
            pltpu.sync_copy(onesv, acc.at[dstbuf.at[j]], add=True)
        return carry

    lax.fori_loop(0, cnt, grp, 0)
    plsc.subcore_barrier()
    pltpu.sync_copy(acc.at[pl.ds(rs, rt)], zb)
    pltpu.sync_copy(zb, out.at[pl.ds(pl.multiple_of(c * n2 + rs, 512), rt)])


def _l1_body(n2, nwin, ei3, g1, zeros4, out,
             table, acc, buf, srcbuf, dstbuf, msg):
    c = lax.axis_index("c")
    s = lax.axis_index("s")
    wid = s * NC + c
    rt = n2 // NS
    rs = pl.multiple_of(s * rt, 512)
    rows = pl.ds(rs, rt)
    pltpu.sync_copy(g1.at[rows, :], buf)
    pltpu.sync_copy(buf, table.at[rows, :])
    pltpu.sync_copy(zeros4.at[rows, :], buf)
    pltpu.sync_copy(buf, acc.at[rows, :])
    plsc.subcore_barrier()
    base, cnt = _part(nwin // LC, NC * NS, wid)

    def grp(g, carry):
        w0 = pl.multiple_of((base + g) * LC, LC)
        pltpu.sync_copy(ei3.at[0, pl.ds(w0, LC), :], srcbuf)
        pltpu.sync_copy(ei3.at[1, pl.ds(w0, LC), :], dstbuf)
        for j in range(LC):
            pltpu.sync_copy(table.at[srcbuf.at[j]], msg)
            pltpu.sync_copy(msg, acc.at[dstbuf.at[j]], add=True)
        return carry

    lax.fori_loop(0, cnt, grp, 0)
    plsc.subcore_barrier()
    pltpu.sync_copy(acc.at[rows, :], buf)
    pltpu.sync_copy(buf, out.at[c, rows, :])


def _l2_body(n2, nwin, ei3, h1g2, out, table, acc, buf, srcbuf, dstbuf, msg):
    c = lax.axis_index("c")
    s = lax.axis_index("s")
    rt = n2 // NS
    rs = pl.multiple_of(s * rt, 512)
    rows = pl.ds(rs, rt)
    pltpu.sync_copy(h1g2.at[c, rows, :], buf)
    pltpu.sync_copy(buf, table.at[rows, :])
    pltpu.sync_copy(buf, acc.at[rows, :])
    plsc.subcore_barrier()
    base, cnt = _part(nwin // LC, NS, s)

    def grp(g, carry):
        w0 = pl.multiple_of((base + g) * LC, LC)
        pltpu.sync_copy(ei3.at[0, pl.ds(w0, LC), :], srcbuf)
        pltpu.sync_copy(ei3.at[1, pl.ds(w0, LC), :], dstbuf)
        for j in range(LC):
            pltpu.sync_copy(table.at[srcbuf.at[j]], msg)
            pltpu.sync_copy(msg, acc.at[dstbuf.at[j]], add=True)
        return carry

    lax.fori_loop(0, cnt, grp, 0)
    plsc.subcore_barrier()
    pltpu.sync_copy(acc.at[rows, :], buf)
    pltpu.sync_copy(buf, out.at[c, rows, :])


def _tc1_body(n, degp0, degp1, x, dinv, g1):
    deg = degp0[0, 0, :] + degp1[0, 0, :] + 1.0
    dv = lax.rsqrt(deg)
    dinv[:, 0] = dv
    g1[:, 0:3] = x[...] * dv[:, None]
    g1[:, 3:4] = jnp.zeros_like(g1[:, 3:4])


def _tc2_body(n, p, g1, dinv, w1, b1, h1g2):
    m = (p[0] + p[1] + g1[...]) * dinv[...]
    h1 = jnp.maximum(
        jax.lax.dot_general(m[:, 0:3], w1[...], (((1,), (0,)), ((), ())),
                            preferred_element_type=jnp.float32) + b1[...],
        0.0,
    )
    h1g = h1 * dinv[...]
    h1g2[0] = h1g[:, 0:8]
    h1g2[1] = h1g[:, 8:16]


def _tc3_body(n, bn, q, dinv, w2, b2, maxv):
    i = pl.program_id(0)
    a0 = q[0] * dinv[...]
    a1 = q[1] * dinv[...]
    h2 = (
        jax.lax.dot_general(a0, w2[0:8, :], (((1,), (0,)), ((), ())),
                            preferred_element_type=jnp.float32)
        + jax.lax.dot_general(a1, w2[8:16, :], (((1,), (0,)), ((), ())),
                              preferred_element_type=jnp.float32)
        + b2[...]
    )
    row = i * bn + lax.broadcasted_iota(jnp.int32, h2.shape, 0)
    h2 = jnp.where(row < n, h2, -jnp.inf)
    bm = jnp.max(h2, axis=0, keepdims=True)

    @pl.when(i == 0)
    def _():
        maxv[...] = jnp.full_like(maxv[...], -jnp.inf)

    maxv[...] = jnp.maximum(maxv[...], bm)


def _tc4_body(maxv, wf, bf, out):
    logits = (
        jax.lax.dot_general(maxv[...], wf[...], (((1,), (0,)), ((), ())),
                            preferred_element_type=jnp.float32)
        + bf[...]
    )
    z = logits - jnp.max(logits)
    out[...] = z - jnp.log(jnp.sum(jnp.exp(z)))


def kernel(x, edge_index, W1, b1, W2, b2, Wf, bf):
    n = x.shape[0]
    e = edge_index.shape[1]
    assert e % (LANE * LC) == 0, "edge count must tile into index windows"
    nwin = e // LANE
    rt_pad = -(-n // NS)
    rt_pad += (-rt_pad) % 16
    n2 = rt_pad * NS
    rt = n2 // NS
    bn = n2 // 16

    ei3 = edge_index.astype(jnp.int32).reshape(2, nwin, LANE)
    x2 = jnp.pad(x, ((0, n2 - n), (0, 0)))
    zeros4 = jnp.zeros((n2, 4), jnp.float32)
    f32 = jnp.float32

    mesh = _sc_mesh()
    deg_k = pl.kernel(
        functools.partial(_deg_body, n2, nwin),
        out_type=jax.ShapeDtypeStruct((NC * n2,), f32),
        mesh=mesh,
        scratch_types=[
            pltpu.VMEM_SHARED((n2,), f32),
            pltpu.VMEM((rt,), f32),
            pltpu.VMEM((LC, LANE), jnp.int32),
            pltpu.VMEM((LANE,), f32),
        ],
    )
    l1_k = pl.kernel(
        functools.partial(_l1_body, n2, nwin),
        out_type=jax.ShapeDtypeStruct((NC, n2, 4), f32),
        mesh=mesh,
        scratch_types=[
            pltpu.VMEM_SHARED((n2, 4), f32),
            pltpu.VMEM_SHARED((n2, 4), f32),
            pltpu.VMEM((rt, 4), f32),
            pltpu.VMEM((LC, LANE), jnp.int32),
            pltpu.VMEM((LC, LANE), jnp.int32),
            pltpu.VMEM((LANE, 4), f32),
        ],
    )
    l2_k = pl.kernel(
        functools.partial(_l2_body, n2, nwin),
        out_type=jax.ShapeDtypeStruct((NC, n2, 8), f32),
        mesh=mesh,
        scratch_types=[
            pltpu.VMEM_SHARED((n2, 8), f32),
            pltpu.VMEM_SHARED((n2, 8), f32),
            pltpu.VMEM((rt, 8), f32),
            pltpu.VMEM((LC, LANE), jnp.int32),
            pltpu.VMEM((LC, LANE), jnp.int32),
            pltpu.VMEM((LANE, 8), f32),
        ],
    )

    degf = deg_k(ei3)

    degp = degf.reshape(NC, n2)
    deg = degp[0, :n] + degp[1, :n] + 1.0
    dinv_ = lax.rsqrt(deg)
    srcv, dstv = edge_index[0], edge_index[1]
    def _layer(h):
        g = h * dinv_[:, None]
        sagg = jax.ops.segment_sum(g[srcv], dstv, num_segments=n) + g
        return dinv_[:, None] * sagg
    h1 = jax.nn.relu(_layer(x) @ W1 + b1)
    h2 = _layer(h1) @ W2 + b2
    logits = jnp.max(h2, axis=0) @ Wf + bf
    return jax.nn.log_softmax(logits)


# speedup vs baseline: 16.0151x; 7.6348x over previous
import functools

import jax
import jax.numpy as jnp
from jax import lax
from jax.experimental import pallas as pl
from jax.experimental.pallas import tpu as pltpu
from jax.experimental.pallas import tpu_sc as plsc

NC = 2
NS = 16
LANE = 128
LC = 16


def _part(nitems, nworkers, wid):
    q, r = nitems // nworkers, nitems % nworkers
    base = wid * q + jnp.minimum(wid, r)
    cnt = q + (wid < r).astype(jnp.int32)
    return base, cnt


def _sc_mesh():
    return plsc.VectorSubcoreMesh(core_axis_name="c", subcore_axis_name="s")


def _zero_fill(zb, nwords):
    def zfill(i, carry):
        zb[pl.ds(pl.multiple_of(i * 16, 16), 16)] = jnp.zeros((16,), jnp.float32)
        return carry

    lax.fori_loop(0, nwords // 16, zfill, 0)


def _deg_body(n2, nwin, ei3, out, acc, zb, dstbuf, onesv):
    c = lax.axis_index("c")
    s = lax.axis_index("s")
    wid = s * NC + c
    rt = n2 // NS
    rs = pl.multiple_of(s * rt, 512)
    _zero_fill(zb, rt)
    for i in range(LANE // 16):
        onesv[pl.ds(i * 16, 16)] = jnp.ones((16,), jnp.float32)
    pltpu.sync_copy(zb, acc.at[pl.ds(rs, rt)])
    plsc.subcore_barrier()
    base, cnt = _part(nwin // LC, NC * NS, wid)

    def grp(g, carry):
        w0 = pl.multiple_of((base + g) * LC, LC)
        pltpu.sync_copy(ei3.at[1, pl.ds(w0, LC), :], dstbuf)

        def win(j, carry2):
            pltpu.sync_copy(onesv, acc.at[dstbuf.at[j]], add=True)
            return carry2

        lax.fori_loop(0, LC, win, 0)
        return carry

    lax.fori_loop(0, cnt, grp, 0)
    plsc.subcore_barrier()
    pltpu.sync_copy(acc.at[pl.ds(rs, rt)], zb)
    pltpu.sync_copy(zb, out.at[pl.ds(pl.multiple_of(c * n2 + rs, 512), rt)])


def _col_body(n2, nwin, ei3, colv, out, tb, ac, zb, srcbuf, dstbuf, msg, gsem):
    c = lax.axis_index("c")
    s = lax.axis_index("s")
    wid = s * NC + c
    rt = n2 // NS
    rs = pl.multiple_of(s * rt, 512)
    rows = pl.ds(rs, rt)
    _zero_fill(zb, rt)
    pltpu.sync_copy(zb, ac.at[rows])
    pltpu.sync_copy(colv.at[rows], zb)
    pltpu.sync_copy(zb, tb.at[rows])
    plsc.subcore_barrier()
    base, cnt = _part(nwin // LC, NC * NS, wid)

    def grp(g, carry):
        w0 = pl.multiple_of((base + g) * LC, LC)
        pltpu.sync_copy(ei3.at[0, pl.ds(w0, LC), :], srcbuf)
        pltpu.sync_copy(ei3.at[1, pl.ds(w0, LC), :], dstbuf)

        def win(j, carry2):
            pltpu.async_copy(tb.at[srcbuf.at[j]], msg, gsem).wait()
            pltpu.sync_copy(msg, ac.at[dstbuf.at[j]], add=True)
            return carry2

        lax.fori_loop(0, LC, win, 0)
        return carry

    lax.fori_loop(0, cnt, grp, 0)
    plsc.subcore_barrier()
    pltpu.sync_copy(ac.at[rows], zb)
    pltpu.sync_copy(zb, out.at[pl.ds(pl.multiple_of(c * n2 + rs, 512), rt)])


def _tc1_body(d0, d1, xt, dinv3, g1t4):
    deg = d0[0, 0, :] + d1[0, 0, :] + 1.0
    dv = lax.rsqrt(deg)
    dinv3[0, 0, :] = dv
    g1t4[...] = xt[...] * dv


def _tc2_body(p6, g1t, dinv3, w1, b1c, out):
    dv = dinv3[0, 0, :]
    mt = (p6[:, 0, 0, 0, :] + p6[:, 1, 0, 0, :] + g1t[:, 0, 0, :]) * dv[None, :]
    h1t = jax.lax.dot_general(w1[...], mt, (((0,), (0,)), ((), ())),
                              preferred_element_type=jnp.float32)
    h1t = jnp.maximum(h1t + b1c[...], 0.0)
    h1gt = h1t * dv[None, :]
    out[...] = h1gt.reshape(out.shape)


def _tc3_body(n, bn, q6, h1g, dinv3, w2, b2r, maxv):
    i = pl.program_id(0)
    at = (q6[:, 0, 0, 0, :] + q6[:, 1, 0, 0, :] + h1g[:, 0, 0, :])
    a = at * dinv3[0, 0, :][None, :]
    h2 = (
        jax.lax.dot_general(a, w2[...], (((0,), (0,)), ((), ())),
                            preferred_element_type=jnp.float32)
        + b2r[...]
    )
    row = i * bn + lax.broadcasted_iota(jnp.int32, h2.shape, 0)
    h2 = jnp.where(row < n, h2, -jnp.inf)
    bm = jnp.max(h2, axis=0, keepdims=True)

    @pl.when(i == 0)
    def _():
        maxv[...] = jnp.full_like(maxv[...], -jnp.inf)

    maxv[...] = jnp.maximum(maxv[...], bm)


def _tc4_body(maxv, wf, bf, out):
    logits = (
        jax.lax.dot_general(maxv[...], wf[...], (((1,), (0,)), ((), ())),
                            preferred_element_type=jnp.float32)
        + bf[...]
    )
    z = logits - jnp.max(logits)
    out[...] = z - jnp.log(jnp.sum(jnp.exp(z)))


def kernel(x, edge_index, W1, b1, W2, b2, Wf, bf):
    n = x.shape[0]
    e = edge_index.shape[1]
    assert e % (LANE * LC) == 0, "edge count must tile into index windows"
    nwin = e // LANE
    rt_pad = -(-n // NS)
    rt_pad += (-rt_pad) % 16
    n2 = rt_pad * NS
    rt = n2 // NS
    bn = n2 // 16

    ei3 = edge_index.astype(jnp.int32).reshape(2, nwin, LANE)
    xt4 = jnp.pad(x, ((0, n2 - n), (0, 0))).T.reshape(3, 16, 1, bn)
    f32 = jnp.float32
    i32 = jnp.int32

    mesh = _sc_mesh()
    deg_k = pl.kernel(
        functools.partial(_deg_body, n2, nwin),
        out_type=jax.ShapeDtypeStruct((NC * n2,), f32),
        mesh=mesh,
        scratch_types=[
            pltpu.VMEM_SHARED((n2,), f32),
            pltpu.VMEM((rt,), f32),
            pltpu.VMEM((LC, LANE), i32),
            pltpu.VMEM((LANE,), f32),
        ],
    )
    col_k = pl.kernel(
        functools.partial(_col_body, n2, nwin),
        out_type=jax.ShapeDtypeStruct((NC * n2,), f32),
        mesh=mesh,
        scratch_types=[
            pltpu.VMEM_SHARED((n2,), f32),
            pltpu.VMEM_SHARED((n2,), f32),
            pltpu.VMEM((rt,), f32),
            pltpu.VMEM((LC, LANE), i32),
            pltpu.VMEM((LC, LANE), i32),
            pltpu.VMEM((LANE,), f32),
            pltpu.SemaphoreType.DMA,
        ],
    )

    degf = deg_k(ei3)
    d0 = degf[:n2].reshape(16, 1, bn)
    d1 = degf[n2:].reshape(16, 1, bn)

    dinv3, g1t4 = pl.pallas_call(
        _tc1_body,
        grid=(16,),
        in_specs=[
            pl.BlockSpec((1, 1, bn), lambda i: (i, 0, 0)),
            pl.BlockSpec((1, 1, bn), lambda i: (i, 0, 0)),
            pl.BlockSpec((3, 1, 1, bn), lambda i: (0, i, 0, 0)),
        ],
        out_specs=[
            pl.BlockSpec((1, 1, bn), lambda i: (i, 0, 0)),
            pl.BlockSpec((3, 1, 1, bn), lambda i: (0, i, 0, 0)),
        ],
        out_shape=[
            jax.ShapeDtypeStruct((16, 1, bn), f32),
            jax.ShapeDtypeStruct((3, 16, 1, bn), f32),
        ],
    )(d0, d1, xt4)

    g1cols = g1t4.reshape(3, n2)
    p6 = jnp.stack([col_k(ei3, g1cols[f]) for f in range(3)])
    p6 = p6.reshape(3, NC, 16, 1, bn)

    h1g4 = pl.pallas_call(
        _tc2_body,
        grid=(16,),
        in_specs=[
            pl.BlockSpec((3, NC, 1, 1, bn), lambda i: (0, 0, i, 0, 0)),
            pl.BlockSpec((3, 1, 1, bn), lambda i: (0, i, 0, 0)),
            pl.BlockSpec((1, 1, bn), lambda i: (i, 0, 0)),
            pl.BlockSpec((3, 16), lambda i: (0, 0)),
            pl.BlockSpec((16, 1), lambda i: (0, 0)),
        ],
        out_specs=pl.BlockSpec((16, 1, 1, bn), lambda i: (0, i, 0, 0)),
        out_shape=jax.ShapeDtypeStruct((16, 16, 1, bn), f32),
    )(p6, g1t4, dinv3, W1, b1.reshape(16, 1))

    h1cols = h1g4.reshape(16, n2)
    q6 = jnp.stack([col_k(ei3, h1cols[f]) for f in range(16)])
    q6 = q6.reshape(16, NC, 16, 1, bn)

    maxv = pl.pallas_call(
        functools.partial(_tc3_body, n, bn),
        grid=(16,),
        in_specs=[
            pl.BlockSpec((16, NC, 1, 1, bn), lambda i: (0, 0, i, 0, 0)),
            pl.BlockSpec((16, 1, 1, bn), lambda i: (0, i, 0, 0)),
            pl.BlockSpec((1, 1, bn), lambda i: (i, 0, 0)),
            pl.BlockSpec((16, 32), lambda i: (0, 0)),
            pl.BlockSpec((1, 32), lambda i: (0, 0)),
        ],
        out_specs=pl.BlockSpec((1, 32), lambda i: (0, 0)),
        out_shape=jax.ShapeDtypeStruct((1, 32), f32),
    )(q6, h1g4, dinv3, W2, b2.reshape(1, 32))

    out = pl.pallas_call(
        _tc4_body,
        in_specs=[
            pl.BlockSpec((1, 32), lambda: (0, 0)),
            pl.BlockSpec((32, 6), lambda: (0, 0)),
            pl.BlockSpec((1, 6), lambda: (0, 0)),
        ],
        out_specs=pl.BlockSpec((1, 6), lambda: (0, 0)),
        out_shape=jax.ShapeDtypeStruct((1, 6), f32),
    )(maxv, Wf, bf.reshape(1, 6))

    return out.reshape(6)


# speedup vs baseline: 22.1048x; 1.3802x over previous
import functools

import jax
import jax.numpy as jnp
from jax import lax
from jax.experimental import pallas as pl
from jax.experimental.pallas import tpu as pltpu
from jax.experimental.pallas import tpu_sc as plsc

NC = 2
NS = 16
LANE = 128
LC = 16


def _part(nitems, nworkers, wid):
    q, r = nitems // nworkers, nitems % nworkers
    base = wid * q + jnp.minimum(wid, r)
    cnt = q + (wid < r).astype(jnp.int32)
    return base, cnt


def _sc_mesh():
    return plsc.VectorSubcoreMesh(core_axis_name="c", subcore_axis_name="s")


def _zero_fill(zb, nwords):
    def zfill(i, carry):
        zb[pl.ds(pl.multiple_of(i * 16, 16), 16)] = jnp.zeros((16,), jnp.float32)
        return carry

    lax.fori_loop(0, nwords // 16, zfill, 0)


def _deg_body(n2, nwin, ei3, out, acc, zb, dstbuf, onesv):
    c = lax.axis_index("c")
    s = lax.axis_index("s")
    wid = s * NC + c
    rt = n2 // NS
    rs = pl.multiple_of(s * rt, 512)
    _zero_fill(zb, rt)
    for i in range(LANE // 16):
        onesv[pl.ds(i * 16, 16)] = jnp.ones((16,), jnp.float32)
    pltpu.sync_copy(zb, acc.at[pl.ds(rs, rt)])
    plsc.subcore_barrier()
    base, cnt = _part(nwin // LC, NC * NS, wid)

    def grp(g, carry):
        w0 = pl.multiple_of((base + g) * LC, LC)
        pltpu.sync_copy(ei3.at[1, pl.ds(w0, LC), :], dstbuf)

        def win(j, carry2):
            pltpu.sync_copy(onesv, acc.at[dstbuf.at[j]], add=True)
            return carry2

        lax.fori_loop(0, LC, win, 0)
        return carry

    lax.fori_loop(0, cnt, grp, 0)
    plsc.subcore_barrier()
    pltpu.sync_copy(acc.at[pl.ds(rs, rt)], zb)
    pltpu.sync_copy(zb, out.at[pl.ds(pl.multiple_of(c * n2 + rs, 512), rt)])


def _col_body(n2, nwin, ei3, colv, out, tb, ac, zb, srcbuf, dstbuf, msg,
              msg2, gsem, ssem):
    c = lax.axis_index("c")
    s = lax.axis_index("s")
    wid = s * NC + c
    rt = n2 // NS
    rs = pl.multiple_of(s * rt, 512)
    rows = pl.ds(rs, rt)
    _zero_fill(zb, rt)
    pltpu.sync_copy(zb, ac.at[rows])
    pltpu.sync_copy(colv.at[rows], zb)
    pltpu.sync_copy(zb, tb.at[rows])
    plsc.subcore_barrier()
    base, cnt = _part(nwin // LC, NC * NS, wid)

    def grp(g, carry):
        w0 = pl.multiple_of((base + g) * LC, LC)
        pltpu.sync_copy(ei3.at[0, pl.ds(w0, LC), :], srcbuf)
        pltpu.sync_copy(ei3.at[1, pl.ds(w0, LC), :], dstbuf)
        prev = None
        for j in range(LC):
            mb = msg if j % 2 == 0 else msg2
            gd = pltpu.async_copy(tb.at[srcbuf.at[j]], mb, gsem)
            if prev is not None:
                prev.wait()
            gd.wait()
            prev = pltpu.async_copy(mb, ac.at[dstbuf.at[j]], ssem, add=True)
        prev.wait()
        return carry

    lax.fori_loop(0, cnt, grp, 0)
    plsc.subcore_barrier()
    pltpu.sync_copy(ac.at[rows], zb)
    pltpu.sync_copy(zb, out.at[pl.ds(pl.multiple_of(c * n2 + rs, 512), rt)])


def _tc1_body(d0, d1, xt, dinv3, g1t4):
    deg = d0[0, 0, :] + d1[0, 0, :] + 1.0
    dv = lax.rsqrt(deg)
    dinv3[0, 0, :] = dv
    g1t4[...] = xt[...] * dv


def _tc2_body(p6, g1t, dinv3, w1, b1c, out):
    dv = dinv3[0, 0, :]
    mt = (p6[:, 0, 0, 0, :] + p6[:, 1, 0, 0, :] + g1t[:, 0, 0, :]) * dv[None, :]
    h1t = jax.lax.dot_general(w1[...], mt, (((0,), (0,)), ((), ())),
                              preferred_element_type=jnp.float32)
    h1t = jnp.maximum(h1t + b1c[...], 0.0)
    h1gt = h1t * dv[None, :]
    out[...] = h1gt.reshape(out.shape)


def _tc3_body(n, bn, q6, h1g, dinv3, w2, b2r, maxv):
    i = pl.program_id(0)
    at = (q6[:, 0, 0, 0, :] + q6[:, 1, 0, 0, :] + h1g[:, 0, 0, :])
    a = at * dinv3[0, 0, :][None, :]
    h2 = (
        jax.lax.dot_general(a, w2[...], (((0,), (0,)), ((), ())),
                            preferred_element_type=jnp.float32)
        + b2r[...]
    )
    row = i * bn + lax.broadcasted_iota(jnp.int32, h2.shape, 0)
    h2 = jnp.where(row < n, h2, -jnp.inf)
    bm = jnp.max(h2, axis=0, keepdims=True)

    @pl.when(i == 0)
    def _():
        maxv[...] = jnp.full_like(maxv[...], -jnp.inf)

    maxv[...] = jnp.maximum(maxv[...], bm)


def _tc4_body(maxv, wf, bf, out):
    logits = (
        jax.lax.dot_general(maxv[...], wf[...], (((1,), (0,)), ((), ())),
                            preferred_element_type=jnp.float32)
        + bf[...]
    )
    z = logits - jnp.max(logits)
    out[...] = z - jnp.log(jnp.sum(jnp.exp(z)))


def kernel(x, edge_index, W1, b1, W2, b2, Wf, bf):
    n = x.shape[0]
    e = edge_index.shape[1]
    assert e % (LANE * LC) == 0, "edge count must tile into index windows"
    nwin = e // LANE
    rt_pad = -(-n // NS)
    rt_pad += (-rt_pad) % 16
    n2 = rt_pad * NS
    rt = n2 // NS
    bn = n2 // 16

    ei3 = edge_index.astype(jnp.int32).reshape(2, nwin, LANE)
    xt4 = jnp.pad(x, ((0, n2 - n), (0, 0))).T.reshape(3, 16, 1, bn)
    f32 = jnp.float32
    i32 = jnp.int32

    mesh = _sc_mesh()
    deg_k = pl.kernel(
        functools.partial(_deg_body, n2, nwin),
        out_type=jax.ShapeDtypeStruct((NC * n2,), f32),
        mesh=mesh,
        scratch_types=[
            pltpu.VMEM_SHARED((n2,), f32),
            pltpu.VMEM((rt,), f32),
            pltpu.VMEM((LC, LANE), i32),
            pltpu.VMEM((LANE,), f32),
        ],
    )
    col_k = pl.kernel(
        functools.partial(_col_body, n2, nwin),
        out_type=jax.ShapeDtypeStruct((NC * n2,), f32),
        mesh=mesh,
        scratch_types=[
            pltpu.VMEM_SHARED((n2,), f32),
            pltpu.VMEM_SHARED((n2,), f32),
            pltpu.VMEM((rt,), f32),
            pltpu.VMEM((LC, LANE), i32),
            pltpu.VMEM((LC, LANE), i32),
            pltpu.VMEM((LANE,), f32),
            pltpu.VMEM((LANE,), f32),
            pltpu.SemaphoreType.DMA,
            pltpu.SemaphoreType.DMA,
        ],
    )

    degf = deg_k(ei3)
    d0 = degf[:n2].reshape(16, 1, bn)
    d1 = degf[n2:].reshape(16, 1, bn)

    dinv3, g1t4 = pl.pallas_call(
        _tc1_body,
        grid=(16,),
        in_specs=[
            pl.BlockSpec((1, 1, bn), lambda i: (i, 0, 0)),
            pl.BlockSpec((1, 1, bn), lambda i: (i, 0, 0)),
            pl.BlockSpec((3, 1, 1, bn), lambda i: (0, i, 0, 0)),
        ],
        out_specs=[
            pl.BlockSpec((1, 1, bn), lambda i: (i, 0, 0)),
            pl.BlockSpec((3, 1, 1, bn), lambda i: (0, i, 0, 0)),
        ],
        out_shape=[
            jax.ShapeDtypeStruct((16, 1, bn), f32),
            jax.ShapeDtypeStruct((3, 16, 1, bn), f32),
        ],
    )(d0, d1, xt4)

    g1cols = g1t4.reshape(3, n2)
    p6 = jnp.stack([col_k(ei3, g1cols[f]) for f in range(3)])
    p6 = p6.reshape(3, NC, 16, 1, bn)

    h1g4 = pl.pallas_call(
        _tc2_body,
        grid=(16,),
        in_specs=[
            pl.BlockSpec((3, NC, 1, 1, bn), lambda i: (0, 0, i, 0, 0)),
            pl.BlockSpec((3, 1, 1, bn), lambda i: (0, i, 0, 0)),
            pl.BlockSpec((1, 1, bn), lambda i: (i, 0, 0)),
            pl.BlockSpec((3, 16), lambda i: (0, 0)),
            pl.BlockSpec((16, 1), lambda i: (0, 0)),
        ],
        out_specs=pl.BlockSpec((16, 1, 1, bn), lambda i: (0, i, 0, 0)),
        out_shape=jax.ShapeDtypeStruct((16, 16, 1, bn), f32),
    )(p6, g1t4, dinv3, W1, b1.reshape(16, 1))

    h1cols = h1g4.reshape(16, n2)
    q6 = jnp.stack([col_k(ei3, h1cols[f]) for f in range(16)])
    q6 = q6.reshape(16, NC, 16, 1, bn)

    maxv = pl.pallas_call(
        functools.partial(_tc3_body, n, bn),
        grid=(16,),
        in_specs=[
            pl.BlockSpec((16, NC, 1, 1, bn), lambda i: (0, 0, i, 0, 0)),
            pl.BlockSpec((16, 1, 1, bn), lambda i: (0, i, 0, 0)),
            pl.BlockSpec((1, 1, bn), lambda i: (i, 0, 0)),
            pl.BlockSpec((16, 32), lambda i: (0, 0)),
            pl.BlockSpec((1, 32), lambda i: (0, 0)),
        ],
        out_specs=pl.BlockSpec((1, 32), lambda i: (0, 0)),
        out_shape=jax.ShapeDtypeStruct((1, 32), f32),
    )(q6, h1g4, dinv3, W2, b2.reshape(1, 32))

    out = pl.pallas_call(
        _tc4_body,
        in_specs=[
            pl.BlockSpec((1, 32), lambda: (0, 0)),
            pl.BlockSpec((32, 6), lambda: (0, 0)),
            pl.BlockSpec((1, 6), lambda: (0, 0)),
        ],
        out_specs=pl.BlockSpec((1, 6), lambda: (0, 0)),
        out_shape=jax.ShapeDtypeStruct((1, 6), f32),
    )(maxv, Wf, bf.reshape(1, 6))

    return out.reshape(6)
